# Initial kernel scaffold; baseline (speedup 1.0000x reference)
import functools
import jax
import jax.numpy as jnp
from jax import lax
from jax.experimental import pallas as pl
from jax.experimental.pallas import tpu as pltpu
from jax.experimental.pallas import tpu_sc as plsc

N = 10000
E = 320000
D = 128
H = 128
C = 40
L = 3
EPS = 1e-5

NC = 2
NS = 16
NW = NC * NS
EPW = E // NW
K = 80
NCHUNK = EPW // K
RPT = N // NS
RZ = 25
DW = 16

_mesh = plsc.VectorSubcoreMesh(core_axis_name="c", subcore_axis_name="s")


def _worker_id():
    return lax.axis_index("s") * NC + lax.axis_index("c")


@functools.partial(
    pl.kernel,
    out_type=(
        jax.ShapeDtypeStruct((NC, N, DW), jnp.float32),
        jax.ShapeDtypeStruct((NC, N, DW), jnp.float32),
    ),
    mesh=_mesh,
    scratch_types=[
        pltpu.VMEM((K,), jnp.int32),
        pltpu.VMEM((K,), jnp.int32),
        pltpu.VMEM((K, DW), jnp.float32),
        pltpu.VMEM((RZ, DW), jnp.float32),
        pltpu.VMEM_SHARED((N, DW), jnp.float32),
        pltpu.VMEM_SHARED((N, DW), jnp.float32),
    ],
)
def _sc_degrees(src_hbm, dst_hbm, dout_hbm, din_hbm,
                sidx, didx, ones_v, zbuf, dout_sh, din_sh):
    c = lax.axis_index("c")
    s = lax.axis_index("s")
    wid = _worker_id()

    for i in range(RZ):
        zbuf[i, :] = jnp.zeros((DW,), jnp.float32)
    for i in range(K):
        ones_v[i, :] = jnp.ones((DW,), jnp.float32)

    for t in range(RPT // RZ):
        off = s * RPT + t * RZ
        pltpu.sync_copy(zbuf, dout_sh.at[pl.ds(off, RZ)])
        pltpu.sync_copy(zbuf, din_sh.at[pl.ds(off, RZ)])
    plsc.subcore_barrier()

    def body(j, _):
        base = pl.multiple_of(wid * EPW + j * K, 8)
        pltpu.sync_copy(src_hbm.at[pl.ds(base, K)], sidx)
        pltpu.sync_copy(dst_hbm.at[pl.ds(base, K)], didx)
        pltpu.sync_copy(ones_v, dout_sh.at[sidx], add=True)
        pltpu.sync_copy(ones_v, din_sh.at[didx], add=True)
        return 0

    lax.fori_loop(0, NCHUNK, body, 0)
    plsc.subcore_barrier()

    for t in range(RPT // RZ):
        off = s * RPT + t * RZ
        pltpu.sync_copy(dout_sh.at[pl.ds(off, RZ)], dout_hbm.at[c, pl.ds(off, RZ)])
        pltpu.sync_copy(din_sh.at[pl.ds(off, RZ)], din_hbm.at[c, pl.ds(off, RZ)])


@functools.partial(
    pl.kernel,
    out_type=jax.ShapeDtypeStruct((NC, N, H), jnp.float32),
    mesh=_mesh,
    scratch_types=[
        pltpu.VMEM((K,), jnp.int32),
        pltpu.VMEM((K,), jnp.int32),
        pltpu.VMEM((K, H), jnp.float32),
        pltpu.VMEM((RZ, H), jnp.float32),
        pltpu.VMEM_SHARED((N, H), jnp.float32),
        pltpu.SemaphoreType.DMA,
    ],
)
def _sc_aggregate(h_hbm, src_hbm, dst_hbm, out_hbm,
                  sidx, didx, rows, zbuf, acc, sem):
    c = lax.axis_index("c")
    s = lax.axis_index("s")
    wid = _worker_id()

    def zrow(i, _):
        for j in range(H // 16):
            zbuf[i, pl.ds(j * 16, 16)] = jnp.zeros((16,), jnp.float32)
        return 0
    lax.fori_loop(0, RZ, zrow, 0)

    for t in range(RPT // RZ):
        off = s * RPT + t * RZ
        pltpu.sync_copy(zbuf, acc.at[pl.ds(off, RZ)])
    plsc.subcore_barrier()

    def body(j, _):
        base = pl.multiple_of(wid * EPW + j * K, 8)
        pltpu.sync_copy(src_hbm.at[pl.ds(base, K)], sidx)
        pltpu.sync_copy(dst_hbm.at[pl.ds(base, K)], didx)
        pltpu.async_copy(h_hbm.at[sidx], rows, sem).wait()
        pltpu.sync_copy(rows, acc.at[didx], add=True)
        return 0

    lax.fori_loop(0, NCHUNK, body, 0)
    plsc.subcore_barrier()

    for t in range(RPT // RZ):
        off = s * RPT + t * RZ
        pltpu.sync_copy(acc.at[pl.ds(off, RZ)], out_hbm.at[c, pl.ds(off, RZ)])


BN_ROWS = 2000


def _scale_from_partials(dp):
    deg = dp[0, :, 0:1] + dp[1, :, 0:1]
    return lax.rsqrt(jnp.maximum(deg, 1.0))


def _tc_in_body(feat_ref, w_ref, b_ref, dout_ref, o_ref):
    h = jnp.dot(feat_ref[...], w_ref[...], preferred_element_type=jnp.float32)
    h = jnp.maximum(h + b_ref[...], 0.0)
    o_ref[...] = h * _scale_from_partials(dout_ref[...])


def _tc_layer_body(p_ref, w_ref, b_ref, din_ref, dout_ref, o_ref):
    agg = (p_ref[0] + p_ref[1]) * _scale_from_partials(din_ref[...])
    h = jnp.dot(agg, w_ref[...], preferred_element_type=jnp.float32)
    h = jnp.maximum(h + b_ref[...], 0.0)
    o_ref[...] = h * _scale_from_partials(dout_ref[...])


def _tc_final_body(p_ref, w_ref, b_ref, din_ref, wo_ref, bo_ref, o_ref):
    agg = (p_ref[0] + p_ref[1]) * _scale_from_partials(din_ref[...])
    h = jnp.dot(agg, w_ref[...], preferred_element_type=jnp.float32)
    h = jnp.maximum(h + b_ref[...], 0.0)
    o_ref[...] = jnp.dot(h, wo_ref[...], preferred_element_type=jnp.float32) + bo_ref[...]


def _row_block(last):
    return pl.BlockSpec((BN_ROWS, last), lambda i: (i, 0))


_full_w = pl.BlockSpec((H, H), lambda i: (0, 0))
_full_b = pl.BlockSpec((1, H), lambda i: (0, 0))
_deg_blk = pl.BlockSpec((NC, BN_ROWS, DW), lambda i: (0, i, 0))
_part_blk = pl.BlockSpec((NC, BN_ROWS, H), lambda i: (0, i, 0))
_grid = (N // BN_ROWS,)


def _tc_in(feat, w, b, dout_p):
    return pl.pallas_call(
        _tc_in_body,
        grid=_grid,
        in_specs=[_row_block(D), _full_w, _full_b, _deg_blk],
        out_specs=_row_block(H),
        out_shape=jax.ShapeDtypeStruct((N, H), jnp.float32),
    )(feat, w, b, dout_p)


def _tc_layer(p, w, b, din_p, dout_p):
    return pl.pallas_call(
        _tc_layer_body,
        grid=_grid,
        in_specs=[_part_blk, _full_w, _full_b, _deg_blk, _deg_blk],
        out_specs=_row_block(H),
        out_shape=jax.ShapeDtypeStruct((N, H), jnp.float32),
    )(p, w, b, din_p, dout_p)


def _tc_final(p, w, b, din_p, wo, bo):
    return pl.pallas_call(
        _tc_final_body,
        grid=_grid,
        in_specs=[_part_blk, _full_w, _full_b, _deg_blk, _full_w, _full_b],
        out_specs=_row_block(H),
        out_shape=jax.ShapeDtypeStruct((N, H), jnp.float32),
    )(p, w, b, din_p, wo, bo)


def kernel(feat, edge_index, W_in, b_in, Wc, bc, W_out, b_out, bn_gamma, bn_beta):
    src = edge_index[0]
    dst = edge_index[1]

    g = bn_gamma / jnp.sqrt(jnp.float32(1.0 + EPS))
    w_in = W_in * g[0][None, :]
    b_in_f = (b_in * g[0] + bn_beta[0])[None, :]
    wc_f = Wc * g[1:][:, None, :]
    bc_f = (bc * g[1:] + bn_beta[1:])[:, None, :]
    wo_pad = jnp.zeros((H, H), jnp.float32).at[:, :C].set(W_out)
    bo_pad = jnp.zeros((1, H), jnp.float32).at[0, :C].set(b_out)

    dout_p, din_p = _sc_degrees(src, dst)

    h = _tc_in(feat, w_in, b_in_f, dout_p)
    for i in range(L - 1):
        p = _sc_aggregate(h, src, dst)
        h = _tc_layer(p, wc_f[i], bc_f[i], din_p, dout_p)
    p = _sc_aggregate(h, src, dst)
    out = _tc_final(p, wc_f[L - 1], bc_f[L - 1], din_p, wo_pad, bo_pad)
    return out[:, :C]


# speedup vs baseline: 4.4310x; 4.4310x over previous
import functools
import jax
import jax.numpy as jnp
from jax import lax
from jax.experimental import pallas as pl
from jax.experimental.pallas import tpu as pltpu
from jax.experimental.pallas import tpu_sc as plsc

N = 10000
E = 320000
D = 128
H = 128
C = 40
L = 3
EPS = 1e-5

NC = 2
NS = 16
NW = NC * NS
EPW = E // NW
K = 80
NCHUNK = EPW // K
RB = 624
RZ = 48
NZ = RB // RZ
TAIL = N - NS * RB
TAIL_OFF = NS * RB
DW = 128

def _worker_id():
    return lax.axis_index("s") * NC + lax.axis_index("c")


def _tile_row_chunks(s):
    chunks = [(pl.multiple_of(s * RB + t * RZ, 8), RZ) for t in range(NZ)]
    return chunks


def _sc_degrees_body(src_hbm, dst_hbm, dout_hbm, din_hbm,
                     idx, ones_v, zbuf, acc):
    c = lax.axis_index("c")
    s = lax.axis_index("s")
    wid = _worker_id()

    def zrow(i, _):
        for j in range(DW // 16):
            zbuf[i, pl.ds(j * 16, 16)] = jnp.zeros((16,), jnp.float32)
        return 0
    lax.fori_loop(0, RZ, zrow, 0)

    def orow(i, _):
        for j in range(DW // 16):
            ones_v[i, pl.ds(j * 16, 16)] = jnp.ones((16,), jnp.float32)
        return 0
    lax.fori_loop(0, K, orow, 0)

    def zero_acc():
        for off, nr in _tile_row_chunks(s):
            pltpu.sync_copy(zbuf, acc.at[pl.ds(off, nr)])

        @pl.when(s == NS - 1)
        def _():
            pltpu.sync_copy(zbuf.at[pl.ds(0, TAIL)], acc.at[pl.ds(TAIL_OFF, TAIL)])

        plsc.subcore_barrier()

    def accumulate(e_hbm):
        def body(j, _):
            base = pl.multiple_of(wid * EPW + j * K, 8)
            pltpu.sync_copy(e_hbm.at[pl.ds(base, K)], idx)
            pltpu.sync_copy(ones_v, acc.at[idx], add=True)
            return 0
        lax.fori_loop(0, NCHUNK, body, 0)
        plsc.subcore_barrier()

    def writeout(o_hbm):
        for off, nr in _tile_row_chunks(s):
            pltpu.sync_copy(acc.at[pl.ds(off, nr)], o_hbm.at[c, pl.ds(off, nr)])

        @pl.when(s == NS - 1)
        def _():
            pltpu.sync_copy(acc.at[pl.ds(TAIL_OFF, TAIL)],
                            o_hbm.at[c, pl.ds(TAIL_OFF, TAIL)])

    zero_acc()
    accumulate(src_hbm)
    writeout(dout_hbm)
    zero_acc()
    accumulate(dst_hbm)
    writeout(din_hbm)


def _sc_aggregate_body(h_hbm, src_hbm, dst_hbm, out_hbm,
                       sidx, didx, rows, zbuf, acc, sem):
    c = lax.axis_index("c")
    s = lax.axis_index("s")
    wid = _worker_id()

    def zrow(i, _):
        for j in range(H // 16):
            zbuf[i, pl.ds(j * 16, 16)] = jnp.zeros((16,), jnp.float32)
        return 0
    lax.fori_loop(0, RZ, zrow, 0)

    for off, nr in _tile_row_chunks(s):
        pltpu.sync_copy(zbuf, acc.at[pl.ds(off, nr)])

    @pl.when(s == NS - 1)
    def _():
        pltpu.sync_copy(zbuf.at[pl.ds(0, TAIL)], acc.at[pl.ds(TAIL_OFF, TAIL)])

    plsc.subcore_barrier()

    def body(j, _):
        base = pl.multiple_of(wid * EPW + j * K, 8)
        pltpu.sync_copy(src_hbm.at[pl.ds(base, K)], sidx)
        pltpu.sync_copy(dst_hbm.at[pl.ds(base, K)], didx)
        pltpu.async_copy(h_hbm.at[sidx], rows, sem).wait()
        pltpu.sync_copy(rows, acc.at[didx], add=True)
        return 0

    lax.fori_loop(0, NCHUNK, body, 0)
    plsc.subcore_barrier()

    for off, nr in _tile_row_chunks(s):
        pltpu.sync_copy(acc.at[pl.ds(off, nr)], out_hbm.at[c, pl.ds(off, nr)])

    @pl.when(s == NS - 1)
    def _():
        pltpu.sync_copy(acc.at[pl.ds(TAIL_OFF, TAIL)],
                        out_hbm.at[c, pl.ds(TAIL_OFF, TAIL)])


@functools.lru_cache(maxsize=None)
def _sc_kernels():
    mesh = plsc.VectorSubcoreMesh(core_axis_name="c", subcore_axis_name="s",
                                  num_cores=NC, num_subcores=NS)
    degrees = pl.kernel(
        _sc_degrees_body,
        out_type=(
            jax.ShapeDtypeStruct((NC, N, DW), jnp.float32),
            jax.ShapeDtypeStruct((NC, N, DW), jnp.float32),
        ),
        mesh=mesh,
        scratch_types=[
            pltpu.VMEM((K,), jnp.int32),
            pltpu.VMEM((K, DW), jnp.float32),
            pltpu.VMEM((RZ, DW), jnp.float32),
            pltpu.VMEM_SHARED((N, DW), jnp.float32),
        ],
    )
    aggregate = pl.kernel(
        _sc_aggregate_body,
        out_type=jax.ShapeDtypeStruct((NC, N, H), jnp.float32),
        mesh=mesh,
        scratch_types=[
            pltpu.VMEM((K,), jnp.int32),
            pltpu.VMEM((K,), jnp.int32),
            pltpu.VMEM((K, H), jnp.float32),
            pltpu.VMEM((RZ, H), jnp.float32),
            pltpu.VMEM_SHARED((N, H), jnp.float32),
            pltpu.SemaphoreType.DMA,
        ],
    )
    return degrees, aggregate


BN_ROWS = 2000


def _scale_from_partials(dp):
    deg = dp[0, :, 0:1] + dp[1, :, 0:1]
    return lax.rsqrt(jnp.maximum(deg, 1.0))


def _tc_in_body(feat_ref, w_ref, b_ref, dout_ref, o_ref):
    h = jnp.dot(feat_ref[...], w_ref[...], preferred_element_type=jnp.float32)
    h = jnp.maximum(h + b_ref[...], 0.0)
    o_ref[...] = h * _scale_from_partials(dout_ref[...])


def _tc_layer_body(p_ref, w_ref, b_ref, din_ref, dout_ref, o_ref):
    agg = (p_ref[0] + p_ref[1]) * _scale_from_partials(din_ref[...])
    h = jnp.dot(agg, w_ref[...], preferred_element_type=jnp.float32)
    h = jnp.maximum(h + b_ref[...], 0.0)
    o_ref[...] = h * _scale_from_partials(dout_ref[...])


def _tc_final_body(p_ref, w_ref, b_ref, din_ref, wo_ref, bo_ref, o_ref):
    agg = (p_ref[0] + p_ref[1]) * _scale_from_partials(din_ref[...])
    h = jnp.dot(agg, w_ref[...], preferred_element_type=jnp.float32)
    h = jnp.maximum(h + b_ref[...], 0.0)
    o_ref[...] = jnp.dot(h, wo_ref[...], preferred_element_type=jnp.float32) + bo_ref[...]


def _row_block(last):
    return pl.BlockSpec((BN_ROWS, last), lambda i: (i, 0))


_full_w = pl.BlockSpec((H, H), lambda i: (0, 0))
_full_b = pl.BlockSpec((1, H), lambda i: (0, 0))
_deg_blk = pl.BlockSpec((NC, BN_ROWS, DW), lambda i: (0, i, 0))
_part_blk = pl.BlockSpec((NC, BN_ROWS, H), lambda i: (0, i, 0))
_grid = (N // BN_ROWS,)


def _tc_in(feat, w, b, dout_p):
    return pl.pallas_call(
        _tc_in_body,
        grid=_grid,
        in_specs=[_row_block(D), _full_w, _full_b, _deg_blk],
        out_specs=_row_block(H),
        out_shape=jax.ShapeDtypeStruct((N, H), jnp.float32),
    )(feat, w, b, dout_p)


def _tc_layer(p, w, b, din_p, dout_p):
    return pl.pallas_call(
        _tc_layer_body,
        grid=_grid,
        in_specs=[_part_blk, _full_w, _full_b, _deg_blk, _deg_blk],
        out_specs=_row_block(H),
        out_shape=jax.ShapeDtypeStruct((N, H), jnp.float32),
    )(p, w, b, din_p, dout_p)


def _tc_final(p, w, b, din_p, wo, bo):
    return pl.pallas_call(
        _tc_final_body,
        grid=_grid,
        in_specs=[_part_blk, _full_w, _full_b, _deg_blk, _full_w, _full_b],
        out_specs=_row_block(H),
        out_shape=jax.ShapeDtypeStruct((N, H), jnp.float32),
    )(p, w, b, din_p, wo, bo)


def kernel(feat, edge_index, W_in, b_in, Wc, bc, W_out, b_out, bn_gamma, bn_beta):
    src = edge_index[0]
    dst = edge_index[1]

    g = bn_gamma / jnp.sqrt(jnp.float32(1.0 + EPS))
    w_in = W_in * g[0][None, :]
    b_in_f = (b_in * g[0] + bn_beta[0])[None, :]
    wc_f = Wc * g[1:][:, None, :]
    bc_f = (bc * g[1:] + bn_beta[1:])[:, None, :]
    wo_pad = jnp.zeros((H, H), jnp.float32).at[:, :C].set(W_out)
    bo_pad = jnp.zeros((1, H), jnp.float32).at[0, :C].set(b_out)

    sc_degrees, sc_aggregate = _sc_kernels()
    dout_p, din_p = sc_degrees(src, dst)

    h = _tc_in(feat, w_in, b_in_f, dout_p)
    for i in range(L - 1):
        p = sc_aggregate(h, src, dst)
        h = _tc_layer(p, wc_f[i], bc_f[i], din_p, dout_p)
    p = sc_aggregate(h, src, dst)
    out = _tc_final(p, wc_f[L - 1], bc_f[L - 1], din_p, wo_pad, bo_pad)
    return out[:, :C]


# speedup vs baseline: 7.9731x; 1.7994x over previous
import functools
import jax
import jax.numpy as jnp
from jax import lax
from jax.experimental import pallas as pl
from jax.experimental.pallas import tpu as pltpu
from jax.experimental.pallas import tpu_sc as plsc

N = 10000
E = 320000
D = 128
H = 128
C = 40
L = 3
EPS = 1e-5

NC = 2
NS = 16
NW = NC * NS
EPW = E // NW
K = 80
NCHUNK = EPW // K
RB = 624
RZ = 16
WZ = 48
TAIL = N - NS * RB
TAIL_OFF = NS * RB
DW = 128

def _worker_id():
    return lax.axis_index("s") * NC + lax.axis_index("c")


def _tile_row_chunks(s, step):
    return [(pl.multiple_of(s * RB + t * step, 8), step)
            for t in range(RB // step)]


def _sc_degrees_body(src_hbm, dst_hbm, dout_hbm, din_hbm,
                     idx_all, ones_v, zbuf, acc, sem):
    c = lax.axis_index("c")
    s = lax.axis_index("s")
    wid = _worker_id()

    def zrow(i, _):
        for j in range(DW // 16):
            zbuf[i, pl.ds(j * 16, 16)] = jnp.zeros((16,), jnp.float32)
        return 0
    lax.fori_loop(0, RZ, zrow, 0)

    def orow(i, _):
        for j in range(DW // 16):
            ones_v[i, pl.ds(j * 16, 16)] = jnp.ones((16,), jnp.float32)
        return 0
    lax.fori_loop(0, K, orow, 0)

    def zero_acc():
        for off, nr in _tile_row_chunks(s, RZ):
            pltpu.sync_copy(zbuf, acc.at[pl.ds(off, nr)])

        @pl.when(s == NS - 1)
        def _():
            pltpu.sync_copy(zbuf.at[pl.ds(0, TAIL)], acc.at[pl.ds(TAIL_OFF, TAIL)])

        plsc.subcore_barrier()

    def accumulate(e_hbm):
        pltpu.sync_copy(e_hbm.at[wid], idx_all)
        B = 8

        def group(g, _):
            for u in range(B):
                j = g * B + u
                pltpu.async_copy(ones_v, acc.at[idx_all.at[j]], sem, add=True)
            for u in range(B):
                j = g * B + u
                pltpu.make_async_copy(ones_v, acc.at[idx_all.at[j]], sem).wait()
            return 0

        nfull = NCHUNK // B
        lax.fori_loop(0, nfull, group, 0)
        for j in range(nfull * B, NCHUNK):
            pltpu.async_copy(ones_v, acc.at[idx_all.at[j]], sem, add=True)
        for j in range(nfull * B, NCHUNK):
            pltpu.make_async_copy(ones_v, acc.at[idx_all.at[j]], sem).wait()
        plsc.subcore_barrier()

    def writeout(o_hbm):
        for off, nr in _tile_row_chunks(s, WZ):
            pltpu.sync_copy(acc.at[pl.ds(off, nr)], o_hbm.at[c, pl.ds(off, nr)])

        @pl.when(s == NS - 1)
        def _():
            pltpu.sync_copy(acc.at[pl.ds(TAIL_OFF, TAIL)],
                            o_hbm.at[c, pl.ds(TAIL_OFF, TAIL)])

    zero_acc()
    accumulate(src_hbm)
    writeout(dout_hbm)
    zero_acc()
    accumulate(dst_hbm)
    writeout(din_hbm)


def _sc_aggregate_body(h_hbm, src_hbm, dst_hbm, out_hbm,
                       sidx, dbuf0, dbuf1, rows0, rows1, zbuf, acc,
                       sem0, sem1, dsem0, dsem1):
    c = lax.axis_index("c")
    s = lax.axis_index("s")
    wid = _worker_id()

    def zrow(i, _):
        for j in range(H // 16):
            zbuf[i, pl.ds(j * 16, 16)] = jnp.zeros((16,), jnp.float32)
        return 0
    lax.fori_loop(0, RZ, zrow, 0)

    pltpu.sync_copy(src_hbm.at[wid], sidx)

    for off, nr in _tile_row_chunks(s, RZ):
        pltpu.sync_copy(zbuf, acc.at[pl.ds(off, nr)])

    @pl.when(s == NS - 1)
    def _():
        pltpu.sync_copy(zbuf.at[pl.ds(0, TAIL)], acc.at[pl.ds(TAIL_OFF, TAIL)])

    plsc.subcore_barrier()

    def gather(j, buf, sem):
        pltpu.async_copy(h_hbm.at[sidx.at[j]], buf, sem)

    def gwait(j, buf, sem):
        pltpu.make_async_copy(h_hbm.at[sidx.at[j]], buf, sem).wait()

    def dload(j, buf, sem):
        base = pl.multiple_of((wid * NCHUNK + j) * K, 8)
        pltpu.async_copy(dst_hbm.at[pl.ds(base, K)], buf, sem)

    def dwait(j, buf, sem):
        base = pl.multiple_of((wid * NCHUNK + j) * K, 8)
        pltpu.make_async_copy(dst_hbm.at[pl.ds(base, K)], buf, sem).wait()

    def scat(buf, dbuf):
        pltpu.sync_copy(buf, acc.at[dbuf], add=True)

    gather(0, rows0, sem0)
    dload(0, dbuf0, dsem0)
    dload(1, dbuf1, dsem1)

    def body(i, _):
        j0 = 2 * i
        gwait(j0, rows0, sem0)
        gather(j0 + 1, rows1, sem1)
        dwait(j0, dbuf0, dsem0)
        scat(rows0, dbuf0)

        @pl.when(j0 + 2 < NCHUNK)
        def _():
            dload(j0 + 2, dbuf0, dsem0)

        gwait(j0 + 1, rows1, sem1)
        gather(j0 + 2, rows0, sem0)
        dwait(j0 + 1, dbuf1, dsem1)
        scat(rows1, dbuf1)

        @pl.when(j0 + 3 < NCHUNK)
        def _():
            dload(j0 + 3, dbuf1, dsem1)

        return 0

    lax.fori_loop(0, (NCHUNK - 1) // 2, body, 0)
    gwait(NCHUNK - 1, rows0, sem0)
    dwait(NCHUNK - 1, dbuf0, dsem0)
    scat(rows0, dbuf0)
    plsc.subcore_barrier()

    for off, nr in _tile_row_chunks(s, WZ):
        pltpu.sync_copy(acc.at[pl.ds(off, nr)], out_hbm.at[c, pl.ds(off, nr)])

    @pl.when(s == NS - 1)
    def _():
        pltpu.sync_copy(acc.at[pl.ds(TAIL_OFF, TAIL)],
                        out_hbm.at[c, pl.ds(TAIL_OFF, TAIL)])


@functools.lru_cache(maxsize=None)
def _sc_kernels():
    mesh = plsc.VectorSubcoreMesh(core_axis_name="c", subcore_axis_name="s",
                                  num_cores=NC, num_subcores=NS)
    degrees = pl.kernel(
        _sc_degrees_body,
        out_type=(
            jax.ShapeDtypeStruct((NC, N, DW), jnp.float32),
            jax.ShapeDtypeStruct((NC, N, DW), jnp.float32),
        ),
        mesh=mesh,
        scratch_types=[
            pltpu.VMEM((NCHUNK, K), jnp.int32),
            pltpu.VMEM((K, DW), jnp.float32),
            pltpu.VMEM((RZ, DW), jnp.float32),
            pltpu.VMEM_SHARED((N, DW), jnp.float32),
            pltpu.SemaphoreType.DMA,
        ],
    )
    aggregate = pl.kernel(
        _sc_aggregate_body,
        out_type=jax.ShapeDtypeStruct((NC, N, H), jnp.float32),
        mesh=mesh,
        scratch_types=[
            pltpu.VMEM((NCHUNK, K), jnp.int32),
            pltpu.VMEM((K,), jnp.int32),
            pltpu.VMEM((K,), jnp.int32),
            pltpu.VMEM((K, H), jnp.float32),
            pltpu.VMEM((K, H), jnp.float32),
            pltpu.VMEM((RZ, H), jnp.float32),
            pltpu.VMEM_SHARED((N, H), jnp.float32),
            pltpu.SemaphoreType.DMA,
            pltpu.SemaphoreType.DMA,
            pltpu.SemaphoreType.DMA,
            pltpu.SemaphoreType.DMA,
        ],
    )
    return degrees, aggregate


BN_ROWS = 2000


def _scale_from_partials(dp):
    deg = dp[0, :, 0:1] + dp[1, :, 0:1]
    return lax.rsqrt(jnp.maximum(deg, 1.0))


def _tc_in_body(feat_ref, w_ref, b_ref, dout_ref, o_ref):
    h = jnp.dot(feat_ref[...], w_ref[...], preferred_element_type=jnp.float32)
    h = jnp.maximum(h + b_ref[...], 0.0)
    o_ref[...] = h * _scale_from_partials(dout_ref[...])


def _tc_layer_body(p_ref, w_ref, b_ref, din_ref, dout_ref, o_ref):
    agg = (p_ref[0] + p_ref[1]) * _scale_from_partials(din_ref[...])
    h = jnp.dot(agg, w_ref[...], preferred_element_type=jnp.float32)
    h = jnp.maximum(h + b_ref[...], 0.0)
    o_ref[...] = h * _scale_from_partials(dout_ref[...])


def _tc_final_body(p_ref, w_ref, b_ref, din_ref, wo_ref, bo_ref, o_ref):
    agg = (p_ref[0] + p_ref[1]) * _scale_from_partials(din_ref[...])
    h = jnp.dot(agg, w_ref[...], preferred_element_type=jnp.float32)
    h = jnp.maximum(h + b_ref[...], 0.0)
    o_ref[...] = jnp.dot(h, wo_ref[...], preferred_element_type=jnp.float32) + bo_ref[...]


def _row_block(last):
    return pl.BlockSpec((BN_ROWS, last), lambda i: (i, 0))


_full_w = pl.BlockSpec((H, H), lambda i: (0, 0))
_full_b = pl.BlockSpec((1, H), lambda i: (0, 0))
_deg_blk = pl.BlockSpec((NC, BN_ROWS, DW), lambda i: (0, i, 0))
_part_blk = pl.BlockSpec((NC, BN_ROWS, H), lambda i: (0, i, 0))
_grid = (N // BN_ROWS,)


def _tc_in(feat, w, b, dout_p):
    return pl.pallas_call(
        _tc_in_body,
        grid=_grid,
        in_specs=[_row_block(D), _full_w, _full_b, _deg_blk],
        out_specs=_row_block(H),
        out_shape=jax.ShapeDtypeStruct((N, H), jnp.float32),
    )(feat, w, b, dout_p)


def _tc_layer(p, w, b, din_p, dout_p):
    return pl.pallas_call(
        _tc_layer_body,
        grid=_grid,
        in_specs=[_part_blk, _full_w, _full_b, _deg_blk, _deg_blk],
        out_specs=_row_block(H),
        out_shape=jax.ShapeDtypeStruct((N, H), jnp.float32),
    )(p, w, b, din_p, dout_p)


def _tc_final(p, w, b, din_p, wo, bo):
    return pl.pallas_call(
        _tc_final_body,
        grid=_grid,
        in_specs=[_part_blk, _full_w, _full_b, _deg_blk, _full_w, _full_b],
        out_specs=_row_block(H),
        out_shape=jax.ShapeDtypeStruct((N, H), jnp.float32),
    )(p, w, b, din_p, wo, bo)


def kernel(feat, edge_index, W_in, b_in, Wc, bc, W_out, b_out, bn_gamma, bn_beta):
    src = edge_index[0]
    dst = edge_index[1]

    g = bn_gamma / jnp.sqrt(jnp.float32(1.0 + EPS))
    w_in = W_in * g[0][None, :]
    b_in_f = (b_in * g[0] + bn_beta[0])[None, :]
    wc_f = Wc * g[1:][:, None, :]
    bc_f = (bc * g[1:] + bn_beta[1:])[:, None, :]
    wo_pad = jnp.zeros((H, H), jnp.float32).at[:, :C].set(W_out)
    bo_pad = jnp.zeros((1, H), jnp.float32).at[0, :C].set(b_out)

    sc_degrees, sc_aggregate = _sc_kernels()
    src_r = src.reshape(NW, NCHUNK, K)
    dst_r = dst.reshape(NW, NCHUNK, K)
    dout_p, din_p = sc_degrees(src_r, dst_r)

    h = _tc_in(feat, w_in, b_in_f, dout_p)
    for i in range(L - 1):
        p = sc_aggregate(h, src_r, dst)
        h = _tc_layer(p, wc_f[i], bc_f[i], din_p, dout_p)
    p = sc_aggregate(h, src_r, dst)
    out = _tc_final(p, wc_f[L - 1], bc_f[L - 1], din_p, wo_pad, bo_pad)
    return out[:, :C]


# speedup vs baseline: 9.3390x; 1.1713x over previous
import functools
import jax
import jax.numpy as jnp
from jax import lax
from jax.experimental import pallas as pl
from jax.experimental.pallas import tpu as pltpu
from jax.experimental.pallas import tpu_sc as plsc

N = 10000
E = 320000
D = 128
H = 128
C = 40
L = 3
EPS = 1e-5

NC = 2
NS = 16
NW = NC * NS
EPW = E // NW
K = 80
NCHUNK = EPW // K
RB = 624
RZ = 16
WZ = 48
TAIL = N - NS * RB
TAIL_OFF = NS * RB
DW = 128
DWS = 16
CB = 128
N_PAD = 10240
NCB = N_PAD // CB

def _worker_id():
    return lax.axis_index("s") * NC + lax.axis_index("c")


def _tile_row_chunks(s, step):
    return [(pl.multiple_of(s * RB + t * step, 8), step)
            for t in range(RB // step)]


def _sc_degrees_body(src_hbm, dst_hbm, dout_hbm, din_hbm,
                     sidx_all, didx_all, hsrc, hdst, colbuf, redbuf, obuf,
                     sh_src, sh_dst):
    c = lax.axis_index("c")
    s = lax.axis_index("s")
    wid = _worker_id()
    zeros16 = jnp.zeros((16,), jnp.float32)
    ones16 = jnp.ones((16,), jnp.float32)

    def zh(i, _):
        hsrc[pl.ds(i * 16, 16)] = zeros16
        hdst[pl.ds(i * 16, 16)] = zeros16
        return 0
    lax.fori_loop(0, N_PAD // 16, zh, 0)

    pltpu.sync_copy(src_hbm.at[wid], sidx_all)
    pltpu.sync_copy(dst_hbm.at[wid], didx_all)

    def hrow(r, _):
        for u in range(K // 16):
            v = sidx_all[r, pl.ds(u * 16, 16)]
            plsc.addupdate_scatter(hsrc, [v], ones16)
            w = didx_all[r, pl.ds(u * 16, 16)]
            plsc.addupdate_scatter(hdst, [w], ones16)
        return 0
    lax.fori_loop(0, NCHUNK, hrow, 0)

    pltpu.sync_copy(hsrc, sh_src.at[s])
    pltpu.sync_copy(hdst, sh_dst.at[s])
    plsc.subcore_barrier()

    def reduce_cols(sh, o_hbm, off, ncols):
        pltpu.sync_copy(sh.at[:, pl.ds(off, ncols)], colbuf.at[:, pl.ds(0, ncols)])
        for t in range(ncols // 16):
            acc16 = colbuf[0, pl.ds(t * 16, 16)]
            for i in range(1, NS):
                acc16 = acc16 + colbuf[i, pl.ds(t * 16, 16)]
            redbuf[pl.ds(t * 16, 16)] = acc16
        for n in range(ncols):
            obuf[n, :] = plsc.load_gather(
                redbuf, [jnp.full((16,), n, jnp.int32)])
        pltpu.sync_copy(obuf.at[pl.ds(0, ncols)], o_hbm.at[c, pl.ds(off, ncols)])

    def rchunk(t, _):
        q = s + NS * t
        off = pl.multiple_of(q * CB, 128)
        reduce_cols(sh_src, dout_hbm, off, CB)
        reduce_cols(sh_dst, din_hbm, off, CB)
        return 0
    lax.fori_loop(0, NCB // NS, rchunk, 0)


def _sc_aggregate_body(h_hbm, src_hbm, dst_hbm, out_hbm,
                       sidx, dbuf0, dbuf1, rows0, rows1, zbuf, acc,
                       sem0, sem1, dsem0, dsem1):
    c = lax.axis_index("c")
    s = lax.axis_index("s")
    wid = _worker_id()

    def zrow(i, _):
        for j in range(H // 16):
            zbuf[i, pl.ds(j * 16, 16)] = jnp.zeros((16,), jnp.float32)
        return 0
    lax.fori_loop(0, RZ, zrow, 0)

    pltpu.sync_copy(src_hbm.at[wid], sidx)

    for off, nr in _tile_row_chunks(s, RZ):
        pltpu.sync_copy(zbuf, acc.at[pl.ds(off, nr)])

    @pl.when(s == NS - 1)
    def _():
        pltpu.sync_copy(zbuf.at[pl.ds(0, TAIL)], acc.at[pl.ds(TAIL_OFF, TAIL)])

    plsc.subcore_barrier()

    def gather(j, buf, sem):
        pltpu.async_copy(h_hbm.at[sidx.at[j]], buf, sem)

    def gwait(j, buf, sem):
        pltpu.make_async_copy(h_hbm.at[sidx.at[j]], buf, sem).wait()

    def dload(j, buf, sem):
        base = pl.multiple_of((wid * NCHUNK + j) * K, 8)
        pltpu.async_copy(dst_hbm.at[pl.ds(base, K)], buf, sem)

    def dwait(j, buf, sem):
        base = pl.multiple_of((wid * NCHUNK + j) * K, 8)
        pltpu.make_async_copy(dst_hbm.at[pl.ds(base, K)], buf, sem).wait()

    def scat(buf, dbuf):
        pltpu.sync_copy(buf, acc.at[dbuf], add=True)

    gather(0, rows0, sem0)
    dload(0, dbuf0, dsem0)
    dload(1, dbuf1, dsem1)

    def body(i, _):
        j0 = 2 * i
        gwait(j0, rows0, sem0)
        gather(j0 + 1, rows1, sem1)
        dwait(j0, dbuf0, dsem0)
        scat(rows0, dbuf0)

        @pl.when(j0 + 2 < NCHUNK)
        def _():
            dload(j0 + 2, dbuf0, dsem0)

        gwait(j0 + 1, rows1, sem1)
        gather(j0 + 2, rows0, sem0)
        dwait(j0 + 1, dbuf1, dsem1)
        scat(rows1, dbuf1)

        @pl.when(j0 + 3 < NCHUNK)
        def _():
            dload(j0 + 3, dbuf1, dsem1)

        return 0

    lax.fori_loop(0, (NCHUNK - 1) // 2, body, 0)
    gwait(NCHUNK - 1, rows0, sem0)
    dwait(NCHUNK - 1, dbuf0, dsem0)
    scat(rows0, dbuf0)
    plsc.subcore_barrier()

    for off, nr in _tile_row_chunks(s, WZ):
        pltpu.sync_copy(acc.at[pl.ds(off, nr)], out_hbm.at[c, pl.ds(off, nr)])

    @pl.when(s == NS - 1)
    def _():
        pltpu.sync_copy(acc.at[pl.ds(TAIL_OFF, TAIL)],
                        out_hbm.at[c, pl.ds(TAIL_OFF, TAIL)])


@functools.lru_cache(maxsize=None)
def _sc_kernels():
    mesh = plsc.VectorSubcoreMesh(core_axis_name="c", subcore_axis_name="s",
                                  num_cores=NC, num_subcores=NS)
    degrees = pl.kernel(
        _sc_degrees_body,
        out_type=(
            jax.ShapeDtypeStruct((NC, N_PAD, DWS), jnp.float32),
            jax.ShapeDtypeStruct((NC, N_PAD, DWS), jnp.float32),
        ),
        mesh=mesh,
        scratch_types=[
            pltpu.VMEM((NCHUNK, K), jnp.int32),
            pltpu.VMEM((NCHUNK, K), jnp.int32),
            pltpu.VMEM((N_PAD,), jnp.float32),
            pltpu.VMEM((N_PAD,), jnp.float32),
            pltpu.VMEM((NS, CB), jnp.float32),
            pltpu.VMEM((CB,), jnp.float32),
            pltpu.VMEM((CB, DWS), jnp.float32),
            pltpu.VMEM_SHARED((NS, N_PAD), jnp.float32),
            pltpu.VMEM_SHARED((NS, N_PAD), jnp.float32),
        ],
        compiler_params=pltpu.CompilerParams(needs_layout_passes=False),
    )
    aggregate = pl.kernel(
        _sc_aggregate_body,
        out_type=jax.ShapeDtypeStruct((NC, N, H), jnp.float32),
        mesh=mesh,
        scratch_types=[
            pltpu.VMEM((NCHUNK, K), jnp.int32),
            pltpu.VMEM((K,), jnp.int32),
            pltpu.VMEM((K,), jnp.int32),
            pltpu.VMEM((K, H), jnp.float32),
            pltpu.VMEM((K, H), jnp.float32),
            pltpu.VMEM((RZ, H), jnp.float32),
            pltpu.VMEM_SHARED((N, H), jnp.float32),
            pltpu.SemaphoreType.DMA,
            pltpu.SemaphoreType.DMA,
            pltpu.SemaphoreType.DMA,
            pltpu.SemaphoreType.DMA,
        ],
    )
    return degrees, aggregate


BN_ROWS = 2000


def _scale_from_partials(dp):
    deg = dp[0, :, 0:1] + dp[1, :, 0:1]
    return lax.rsqrt(jnp.maximum(deg, 1.0))


def _tc_in_body(feat_ref, w_ref, b_ref, dout_ref, o_ref):
    h = jnp.dot(feat_ref[...], w_ref[...], preferred_element_type=jnp.float32)
    h = jnp.maximum(h + b_ref[...], 0.0)
    o_ref[...] = h * _scale_from_partials(dout_ref[...])


def _tc_layer_body(p_ref, w_ref, b_ref, din_ref, dout_ref, o_ref):
    agg = (p_ref[0] + p_ref[1]) * _scale_from_partials(din_ref[...])
    h = jnp.dot(agg, w_ref[...], preferred_element_type=jnp.float32)
    h = jnp.maximum(h + b_ref[...], 0.0)
    o_ref[...] = h * _scale_from_partials(dout_ref[...])


def _tc_final_body(p_ref, w_ref, b_ref, din_ref, wo_ref, bo_ref, o_ref):
    agg = (p_ref[0] + p_ref[1]) * _scale_from_partials(din_ref[...])
    h = jnp.dot(agg, w_ref[...], preferred_element_type=jnp.float32)
    h = jnp.maximum(h + b_ref[...], 0.0)
    o_ref[...] = jnp.dot(h, wo_ref[...], preferred_element_type=jnp.float32) + bo_ref[...]


def _row_block(last):
    return pl.BlockSpec((BN_ROWS, last), lambda i: (i, 0))


_full_w = pl.BlockSpec((H, H), lambda i: (0, 0))
_full_b = pl.BlockSpec((1, H), lambda i: (0, 0))
_deg_blk = pl.BlockSpec((NC, BN_ROWS, DWS), lambda i: (0, i, 0))
_part_blk = pl.BlockSpec((NC, BN_ROWS, H), lambda i: (0, i, 0))
_grid = (N // BN_ROWS,)


def _tc_in(feat, w, b, dout_p):
    return pl.pallas_call(
        _tc_in_body,
        grid=_grid,
        in_specs=[_row_block(D), _full_w, _full_b, _deg_blk],
        out_specs=_row_block(H),
        out_shape=jax.ShapeDtypeStruct((N, H), jnp.float32),
    )(feat, w, b, dout_p)


def _tc_layer(p, w, b, din_p, dout_p):
    return pl.pallas_call(
        _tc_layer_body,
        grid=_grid,
        in_specs=[_part_blk, _full_w, _full_b, _deg_blk, _deg_blk],
        out_specs=_row_block(H),
        out_shape=jax.ShapeDtypeStruct((N, H), jnp.float32),
    )(p, w, b, din_p, dout_p)


def _tc_final(p, w, b, din_p, wo, bo):
    return pl.pallas_call(
        _tc_final_body,
        grid=_grid,
        in_specs=[_part_blk, _full_w, _full_b, _deg_blk, _full_w, _full_b],
        out_specs=_row_block(H),
        out_shape=jax.ShapeDtypeStruct((N, H), jnp.float32),
    )(p, w, b, din_p, wo, bo)


def kernel(feat, edge_index, W_in, b_in, Wc, bc, W_out, b_out, bn_gamma, bn_beta):
    src = edge_index[0]
    dst = edge_index[1]

    g = bn_gamma / jnp.sqrt(jnp.float32(1.0 + EPS))
    w_in = W_in * g[0][None, :]
    b_in_f = (b_in * g[0] + bn_beta[0])[None, :]
    wc_f = Wc * g[1:][:, None, :]
    bc_f = (bc * g[1:] + bn_beta[1:])[:, None, :]
    wo_pad = jnp.zeros((H, H), jnp.float32).at[:, :C].set(W_out)
    bo_pad = jnp.zeros((1, H), jnp.float32).at[0, :C].set(b_out)

    sc_degrees, sc_aggregate = _sc_kernels()
    src_r = src.reshape(NW, NCHUNK, K)
    dst_r = dst.reshape(NW, NCHUNK, K)
    dout_p, din_p = sc_degrees(src_r, dst_r)
    dout_p = dout_p[:, :N]
    din_p = din_p[:, :N]

    h = _tc_in(feat, w_in, b_in_f, dout_p)
    for i in range(L - 1):
        p = sc_aggregate(h, src_r, dst)
        h = _tc_layer(p, wc_f[i], bc_f[i], din_p, dout_p)
    p = sc_aggregate(h, src_r, dst)
    out = _tc_final(p, wc_f[L - 1], bc_f[L - 1], din_p, wo_pad, bo_pad)
    return out[:, :C]


# speedup vs baseline: 9.3427x; 1.0004x over previous
import functools
import jax
import jax.numpy as jnp
from jax import lax
from jax.experimental import pallas as pl
from jax.experimental.pallas import tpu as pltpu
from jax.experimental.pallas import tpu_sc as plsc

N = 10000
E = 320000
D = 128
H = 128
C = 40
L = 3
EPS = 1e-5

NC = 2
NS = 16
NW = NC * NS
EPW = E // NW
K = 80
NCHUNK = EPW // K
RB = 624
RZ = 16
WZ = 48
TAIL = N - NS * RB
TAIL_OFF = NS * RB
DWS = 16
CB = 128
N_PAD = 10240
NCB = N_PAD // CB

def _worker_id():
    return lax.axis_index("s") * NC + lax.axis_index("c")


def _tile_row_chunks(s, step):
    return [(pl.multiple_of(s * RB + t * step, 8), step)
            for t in range(RB // step)]


def _sc_degrees_body(src_hbm, dst_hbm, dout_hbm, din_hbm,
                     sidx_all, didx_all, hsrc, hdst, colbuf, redbuf, obuf,
                     sh_src, sh_dst):
    c = lax.axis_index("c")
    s = lax.axis_index("s")
    wid = _worker_id()
    zeros16 = jnp.zeros((16,), jnp.float32)
    ones16 = jnp.ones((16,), jnp.float32)

    def zh(i, _):
        hsrc[pl.ds(i * 16, 16)] = zeros16
        hdst[pl.ds(i * 16, 16)] = zeros16
        return 0
    lax.fori_loop(0, N_PAD // 16, zh, 0)

    pltpu.sync_copy(src_hbm.at[wid], sidx_all)
    pltpu.sync_copy(dst_hbm.at[wid], didx_all)

    def hrow(r, _):
        for u in range(K // 16):
            v = sidx_all[r, pl.ds(u * 16, 16)]
            plsc.addupdate_scatter(hsrc, [v], ones16)
            w = didx_all[r, pl.ds(u * 16, 16)]
            plsc.addupdate_scatter(hdst, [w], ones16)
        return 0
    lax.fori_loop(0, NCHUNK, hrow, 0)

    pltpu.sync_copy(hsrc, sh_src.at[s])
    pltpu.sync_copy(hdst, sh_dst.at[s])
    plsc.subcore_barrier()

    def reduce_cols(sh, o_hbm, off, ncols):
        pltpu.sync_copy(sh.at[:, pl.ds(off, ncols)], colbuf.at[:, pl.ds(0, ncols)])
        for t in range(ncols // 16):
            acc16 = colbuf[0, pl.ds(t * 16, 16)]
            for i in range(1, NS):
                acc16 = acc16 + colbuf[i, pl.ds(t * 16, 16)]
            redbuf[pl.ds(t * 16, 16)] = acc16
        for n in range(ncols):
            obuf[n, :] = plsc.load_gather(
                redbuf, [jnp.full((16,), n, jnp.int32)])
        pltpu.sync_copy(obuf.at[pl.ds(0, ncols)], o_hbm.at[c, pl.ds(off, ncols)])

    def rchunk(t, _):
        q = s + NS * t
        off = pl.multiple_of(q * CB, 128)
        reduce_cols(sh_src, dout_hbm, off, CB)
        reduce_cols(sh_dst, din_hbm, off, CB)
        return 0
    lax.fori_loop(0, NCB // NS, rchunk, 0)


def _sc_aggregate_body(h_hbm, src_hbm, dst_hbm, out_hbm,
                       sidx, dbuf0, dbuf1, rows0, rows1, zbuf, acc,
                       sem0, sem1, dsem0, dsem1):
    c = lax.axis_index("c")
    s = lax.axis_index("s")
    wid = _worker_id()

    def zrow(i, _):
        for j in range(H // 16):
            zbuf[i, pl.ds(j * 16, 16)] = jnp.zeros((16,), jnp.float32)
        return 0
    lax.fori_loop(0, RZ, zrow, 0)

    pltpu.sync_copy(src_hbm.at[wid], sidx)

    for off, nr in _tile_row_chunks(s, RZ):
        pltpu.sync_copy(zbuf, acc.at[pl.ds(off, nr)])

    @pl.when(s == NS - 1)
    def _():
        pltpu.sync_copy(zbuf.at[pl.ds(0, TAIL)], acc.at[pl.ds(TAIL_OFF, TAIL)])

    plsc.subcore_barrier()

    def gather(j, buf, sem):
        pltpu.async_copy(h_hbm.at[sidx.at[j]], buf, sem)

    def gwait(j, buf, sem):
        pltpu.make_async_copy(h_hbm.at[sidx.at[j]], buf, sem).wait()

    def dload(j, buf, sem):
        base = pl.multiple_of((wid * NCHUNK + j) * K, 8)
        pltpu.async_copy(dst_hbm.at[pl.ds(base, K)], buf, sem)

    def dwait(j, buf, sem):
        base = pl.multiple_of((wid * NCHUNK + j) * K, 8)
        pltpu.make_async_copy(dst_hbm.at[pl.ds(base, K)], buf, sem).wait()

    def scat(buf, dbuf):
        pltpu.sync_copy(buf, acc.at[dbuf], add=True)

    gather(0, rows0, sem0)
    dload(0, dbuf0, dsem0)
    dload(1, dbuf1, dsem1)

    def body(i, _):
        j0 = 2 * i
        gwait(j0, rows0, sem0)
        gather(j0 + 1, rows1, sem1)
        dwait(j0, dbuf0, dsem0)
        scat(rows0, dbuf0)

        @pl.when(j0 + 2 < NCHUNK)
        def _():
            dload(j0 + 2, dbuf0, dsem0)

        gwait(j0 + 1, rows1, sem1)
        gather(j0 + 2, rows0, sem0)
        dwait(j0 + 1, dbuf1, dsem1)
        scat(rows1, dbuf1)

        @pl.when(j0 + 3 < NCHUNK)
        def _():
            dload(j0 + 3, dbuf1, dsem1)

        return 0

    lax.fori_loop(0, (NCHUNK - 1) // 2, body, 0)
    gwait(NCHUNK - 1, rows0, sem0)
    dwait(NCHUNK - 1, dbuf0, dsem0)
    scat(rows0, dbuf0)
    plsc.subcore_barrier()

    for off, nr in _tile_row_chunks(s, WZ):
        pltpu.sync_copy(acc.at[pl.ds(off, nr)], out_hbm.at[c, pl.ds(off, nr)])

    @pl.when(s == NS - 1)
    def _():
        pltpu.sync_copy(acc.at[pl.ds(TAIL_OFF, TAIL)],
                        out_hbm.at[c, pl.ds(TAIL_OFF, TAIL)])


@functools.lru_cache(maxsize=None)
def _sc_kernels():
    mesh = plsc.VectorSubcoreMesh(core_axis_name="c", subcore_axis_name="s",
                                  num_cores=NC, num_subcores=NS)
    degrees = pl.kernel(
        _sc_degrees_body,
        out_type=(
            jax.ShapeDtypeStruct((NC, N_PAD, DWS), jnp.float32),
            jax.ShapeDtypeStruct((NC, N_PAD, DWS), jnp.float32),
        ),
        mesh=mesh,
        scratch_types=[
            pltpu.VMEM((NCHUNK, K), jnp.int32),
            pltpu.VMEM((NCHUNK, K), jnp.int32),
            pltpu.VMEM((N_PAD,), jnp.float32),
            pltpu.VMEM((N_PAD,), jnp.float32),
            pltpu.VMEM((NS, CB), jnp.float32),
            pltpu.VMEM((CB,), jnp.float32),
            pltpu.VMEM((CB, DWS), jnp.float32),
            pltpu.VMEM_SHARED((NS, N_PAD), jnp.float32),
            pltpu.VMEM_SHARED((NS, N_PAD), jnp.float32),
        ],
        compiler_params=pltpu.CompilerParams(needs_layout_passes=False),
    )
    aggregate = pl.kernel(
        _sc_aggregate_body,
        out_type=jax.ShapeDtypeStruct((NC, N, H), jnp.float32),
        mesh=mesh,
        scratch_types=[
            pltpu.VMEM((NCHUNK, K), jnp.int32),
            pltpu.VMEM((K,), jnp.int32),
            pltpu.VMEM((K,), jnp.int32),
            pltpu.VMEM((K, H), jnp.float32),
            pltpu.VMEM((K, H), jnp.float32),
            pltpu.VMEM((RZ, H), jnp.float32),
            pltpu.VMEM_SHARED((N, H), jnp.float32),
            pltpu.SemaphoreType.DMA,
            pltpu.SemaphoreType.DMA,
            pltpu.SemaphoreType.DMA,
            pltpu.SemaphoreType.DMA,
        ],
    )
    return degrees, aggregate


BN_ROWS = 2000


def _scale_from_partials(dp):
    deg = dp[0, :, 0:1] + dp[1, :, 0:1]
    return lax.rsqrt(jnp.maximum(deg, 1.0))


def _tc_in_body(feat_ref, w_ref, b_ref, dout_ref, o_ref):
    h = jnp.dot(feat_ref[...], w_ref[...], preferred_element_type=jnp.float32)
    h = jnp.maximum(h + b_ref[...], 0.0)
    o_ref[...] = h * _scale_from_partials(dout_ref[...])


def _tc_layer_body(p_ref, w_ref, b_ref, din_ref, dout_ref, o_ref):
    agg = (p_ref[0] + p_ref[1]) * _scale_from_partials(din_ref[...])
    h = jnp.dot(agg, w_ref[...], preferred_element_type=jnp.float32)
    h = jnp.maximum(h + b_ref[...], 0.0)
    o_ref[...] = h * _scale_from_partials(dout_ref[...])


def _tc_final_body(p_ref, w_ref, b_ref, din_ref, wo_ref, bo_ref, o_ref):
    agg = (p_ref[0] + p_ref[1]) * _scale_from_partials(din_ref[...])
    h = jnp.dot(agg, w_ref[...], preferred_element_type=jnp.float32)
    h = jnp.maximum(h + b_ref[...], 0.0)
    o_ref[...] = jnp.dot(h, wo_ref[...], preferred_element_type=jnp.float32) + bo_ref[...]


def _row_block(last):
    return pl.BlockSpec((BN_ROWS, last), lambda i: (i, 0))


_full_w = pl.BlockSpec((H, H), lambda i: (0, 0))
_full_b = pl.BlockSpec((1, H), lambda i: (0, 0))
_deg_blk = pl.BlockSpec((NC, BN_ROWS, DWS), lambda i: (0, i, 0))
_part_blk = pl.BlockSpec((NC, BN_ROWS, H), lambda i: (0, i, 0))
_grid = (N // BN_ROWS,)


def _tc_in(feat, w, b, dout_p):
    return pl.pallas_call(
        _tc_in_body,
        grid=_grid,
        in_specs=[_row_block(D), _full_w, _full_b, _deg_blk],
        out_specs=_row_block(H),
        out_shape=jax.ShapeDtypeStruct((N, H), jnp.float32),
    )(feat, w, b, dout_p)


def _tc_layer(p, w, b, din_p, dout_p):
    return pl.pallas_call(
        _tc_layer_body,
        grid=_grid,
        in_specs=[_part_blk, _full_w, _full_b, _deg_blk, _deg_blk],
        out_specs=_row_block(H),
        out_shape=jax.ShapeDtypeStruct((N, H), jnp.float32),
    )(p, w, b, din_p, dout_p)


def _tc_final(p, w, b, din_p, wo, bo):
    return pl.pallas_call(
        _tc_final_body,
        grid=_grid,
        in_specs=[_part_blk, _full_w, _full_b, _deg_blk, _full_w, _full_b],
        out_specs=_row_block(H),
        out_shape=jax.ShapeDtypeStruct((N, H), jnp.float32),
    )(p, w, b, din_p, wo, bo)


def kernel(feat, edge_index, W_in, b_in, Wc, bc, W_out, b_out, bn_gamma, bn_beta):
    src = edge_index[0]
    dst = edge_index[1]

    g = bn_gamma / jnp.sqrt(jnp.float32(1.0 + EPS))
    w_in = W_in * g[0][None, :]
    b_in_f = (b_in * g[0] + bn_beta[0])[None, :]
    wc_f = Wc * g[1:][:, None, :]
    bc_f = (bc * g[1:] + bn_beta[1:])[:, None, :]
    wo_pad = jnp.zeros((H, H), jnp.float32).at[:, :C].set(W_out)
    bo_pad = jnp.zeros((1, H), jnp.float32).at[0, :C].set(b_out)

    sc_degrees, sc_aggregate = _sc_kernels()
    src_r = src.reshape(NW, NCHUNK, K)
    dst_r = dst.reshape(NW, NCHUNK, K)
    dout_p, din_p = sc_degrees(src_r, dst_r)
    dout_p = dout_p[:, :N]
    din_p = din_p[:, :N]

    h = _tc_in(feat, w_in, b_in_f, dout_p)
    for i in range(L - 1):
        p = sc_aggregate(h, src_r, dst)
        h = _tc_layer(p, wc_f[i], bc_f[i], din_p, dout_p)
    p = sc_aggregate(h, src_r, dst)
    out = _tc_final(p, wc_f[L - 1], bc_f[L - 1], din_p, wo_pad, bo_pad)
    return out[:, :C]
